# baseline (speedup 1.0000x reference)
import functools

import jax
import jax.numpy as jnp
from jax import lax
from jax.experimental import pallas as pl
from jax.experimental.pallas import tpu as pltpu
from jax.experimental.pallas import tpu_sc as plsc

B = 64
P = 576
D = 768

_NC = 2
_NS = 16
_NW = _NC * _NS
_NSTRIPE = 8
_SPP = P // _NSTRIPE
_NBG = _NW // _NSTRIPE
_BPG = B // _NBG
_CPB = 24
_CPS = _SPP // _CPB
_NCH = _BPG * _CPS
_GPR = D // 16


def _sc_body(x_hbm, t_hbm, o_hbm, posv,
             xb0, xb1, xb2, xb3,
             psem, is0, is1, is2, is3, os0, os1, os2, os3):
    cid = lax.axis_index("c")
    sid = lax.axis_index("s")
    wid = sid * _NC + cid
    sp = wid % _NSTRIPE
    bg = wid // _NSTRIPE
    p0 = sp * _SPP
    b0 = bg * _BPG

    pltpu.make_async_copy(t_hbm.at[pl.ds(p0, _SPP), :], posv, psem).start()

    xbufs = (xb0, xb1, xb2, xb3)
    isems = (is0, is1, is2, is3)
    osems = (os0, os1, os2, os3)

    def xslice(c):
        b = b0 + c // _CPS
        j = c % _CPS
        return (b, pl.ds(p0 + j * _CPB, _CPB))

    def start_in(c, sx):
        b, rows = xslice(c)
        pltpu.make_async_copy(x_hbm.at[b, rows, :], xbufs[sx], isems[sx]).start()

    start_in(0, 0)
    start_in(1, 1)

    pltpu.make_async_copy(t_hbm.at[pl.ds(p0, _SPP), :], posv, psem).wait()

    def step(c, sx):
        xbuf, isem, osem = xbufs[sx], isems[sx], osems[sx]
        b, rows = xslice(c)
        jrow = (c % _CPS) * _CPB

        pltpu.make_async_copy(x_hbm.at[b, rows, :], xbuf, isem).wait()

        @plsc.parallel_loop(0, _CPB * _GPR, 1, unroll=8)
        def _add_group(i):
            r = i // _GPR
            g = (i % _GPR) * 16
            plsc.addupdate(xbuf.at[r, pl.ds(g, 16)],
                           posv[jrow + r, pl.ds(g, 16)])

        pltpu.make_async_copy(xbuf, o_hbm.at[b, rows, :], osem).start()

        sx2 = (sx + 2) % 4

        @pl.when(c + 2 < _NCH)
        def _():
            @pl.when(c >= 2)
            def _():
                bp, rowsp = xslice(c - 2)
                pltpu.make_async_copy(
                    xbufs[sx2], o_hbm.at[bp, rowsp, :], osems[sx2]).wait()

            start_in(c + 2, sx2)

    def loop(i, carry):
        c0 = i * 4
        step(c0, 0)
        step(c0 + 1, 1)
        step(c0 + 2, 2)
        step(c0 + 3, 3)
        return carry

    lax.fori_loop(0, _NCH // 4, loop, 0)

    for k in range(4):
        c = _NCH - 4 + k
        b, rows = xslice(c)
        pltpu.make_async_copy(
            xbufs[c % 4], o_hbm.at[b, rows, :], osems[c % 4]).wait()


def kernel(encoded_patches, pos_table):
    mesh = plsc.VectorSubcoreMesh(core_axis_name="c", subcore_axis_name="s")
    k = functools.partial(
        pl.kernel,
        mesh=mesh,
        out_type=jax.ShapeDtypeStruct((B, P, D), jnp.float32),
        scratch_types=[
            pltpu.VMEM((_SPP, D), jnp.float32),
            pltpu.VMEM((_CPB, D), jnp.float32),
            pltpu.VMEM((_CPB, D), jnp.float32),
            pltpu.VMEM((_CPB, D), jnp.float32),
            pltpu.VMEM((_CPB, D), jnp.float32),
            pltpu.SemaphoreType.DMA,
            pltpu.SemaphoreType.DMA,
            pltpu.SemaphoreType.DMA,
            pltpu.SemaphoreType.DMA,
            pltpu.SemaphoreType.DMA,
            pltpu.SemaphoreType.DMA,
            pltpu.SemaphoreType.DMA,
            pltpu.SemaphoreType.DMA,
            pltpu.SemaphoreType.DMA,
        ],
    )(_sc_body)
    return k(encoded_patches, pos_table)


# speedup vs baseline: 1.2742x; 1.2742x over previous
import functools

import jax
import jax.numpy as jnp
from jax import lax
from jax.experimental import pallas as pl
from jax.experimental.pallas import tpu as pltpu
from jax.experimental.pallas import tpu_sc as plsc

B = 64
P = 576
D = 768

_NC = 2
_NS = 16
_NW = _NC * _NS
_RPW = (B * P) // _NW
_CPB = 8
_NCH = _RPW // _CPB
_GPR = D // 16
_XR = 6
_PR = 3
_PD = 3


def _sc_body(x_hbm, t_hbm, o_hbm, spos,
             pb0, pb1, pb2, xb0, xb1, xb2, xb3, xb4, xb5,
             ps0, ps1, ps2,
             is0, is1, is2, is3, is4, is5,
             os0, os1, os2, os3, os4, os5):
    cid = lax.axis_index("c")
    sid = lax.axis_index("s")
    wid = sid * _NC + cid
    row0 = wid * _RPW

    @pl.when(sid == 0)
    def _():
        pltpu.sync_copy(t_hbm, spos)

    plsc.subcore_barrier()

    xbufs = (xb0, xb1, xb2, xb3, xb4, xb5)
    isems = (is0, is1, is2, is3, is4, is5)
    osems = (os0, os1, os2, os3, os4, os5)
    pbufs = (pb0, pb1, pb2)
    psems = (ps0, ps1, ps2)

    def xrow(c):
        return row0 + c * _CPB

    def prow(c):
        return (c % (P // _CPB)) * _CPB

    def start_in(c, sx, sp):
        pltpu.make_async_copy(
            x_hbm.at[pl.ds(xrow(c), _CPB), :], xbufs[sx], isems[sx]).start()
        pltpu.make_async_copy(
            spos.at[pl.ds(prow(c), _CPB), :], pbufs[sp], psems[sp]).start()

    start_in(0, 0, 0)
    start_in(1, 1, 1)
    start_in(2, 2, 2)

    def step(c, sx, sp):
        xbuf, isem, osem = xbufs[sx], isems[sx], osems[sx]
        pbuf, psem = pbufs[sp], psems[sp]

        pltpu.make_async_copy(
            x_hbm.at[pl.ds(xrow(c), _CPB), :], xbuf, isem).wait()
        pltpu.make_async_copy(
            spos.at[pl.ds(prow(c), _CPB), :], pbuf, psem).wait()

        @plsc.parallel_loop(0, _CPB * _GPR, 1, unroll=8)
        def _add_group(i):
            r = i // _GPR
            g = (i % _GPR) * 16
            plsc.addupdate(xbuf.at[r, pl.ds(g, 16)], pbuf[r, pl.ds(g, 16)])

        pltpu.make_async_copy(
            xbuf, o_hbm.at[pl.ds(xrow(c), _CPB), :], osem).start()

        sx2 = (sx + _PD) % _XR

        @pl.when(c + _PD < _NCH)
        def _():
            @pl.when(c + _PD >= _XR)
            def _():
                cp = c + _PD - _XR
                pltpu.make_async_copy(
                    xbufs[sx2],
                    o_hbm.at[pl.ds(xrow(cp), _CPB), :],
                    osems[sx2]).wait()

            start_in(c + _PD, sx2, sp)

    def loop(i, carry):
        c0 = i * _XR
        for s in range(_XR):
            step(c0 + s, s, s % _PR)
        return carry

    lax.fori_loop(0, _NCH // _XR, loop, 0)

    for k in range(_XR):
        c = _NCH - _XR + k
        pltpu.make_async_copy(
            xbufs[c % _XR], o_hbm.at[pl.ds(xrow(c), _CPB), :],
            osems[c % _XR]).wait()


def kernel(encoded_patches, pos_table):
    x2 = encoded_patches.reshape(B * P, D)
    mesh = plsc.VectorSubcoreMesh(core_axis_name="c", subcore_axis_name="s")
    k = functools.partial(
        pl.kernel,
        mesh=mesh,
        out_type=jax.ShapeDtypeStruct((B * P, D), jnp.float32),
        scratch_types=(
            [pltpu.VMEM_SHARED((P, D), jnp.float32)]
            + [pltpu.VMEM((_CPB, D), jnp.float32)] * (_PR + _XR)
            + [pltpu.SemaphoreType.DMA] * (_PR + 2 * _XR)
        ),
    )(_sc_body)
    out = k(x2, pos_table)
    return out.reshape(B, P, D)
